# baseline (speedup 1.0000x reference)
import jax
import jax.numpy as jnp
from jax.experimental import pallas as pl
from jax.experimental.pallas import tpu as pltpu

N = 100000
D = 128
K = 16
OUT = 256
TN = 5000
CORES = 2


def _pass1(x_ref, ev_ref, part_ref):
    j = pl.program_id(1)

    @pl.when(j == 0)
    def _():
        part_ref[...] = jnp.zeros_like(part_ref)

    part_ref[0] += jax.lax.dot_general(
        ev_ref[...], x_ref[...],
        dimension_numbers=(((0,), (0,)), ((), ())),
        preferred_element_type=jnp.float32)


def _pass2(part_ref, evals_ref, mask_ref, w1t_ref, b1_ref, w2t_ref,
           b2_ref, wp_ref, ev_ref, bp_ref, out_ref):
    xfreq = part_ref[0] + part_ref[1]
    h = jnp.maximum(
        jnp.dot(w1t_ref[...], evals_ref[...],
                preferred_element_type=jnp.float32) + b1_ref[...], 0.0)
    filt = jnp.tanh(
        jnp.dot(w2t_ref[...], h,
                preferred_element_type=jnp.float32) + b2_ref[...])
    filt = filt * mask_ref[...]
    m = jnp.dot(filt * xfreq, wp_ref[...],
                preferred_element_type=jnp.float32)
    out_ref[...] = jnp.dot(ev_ref[...], m,
                           preferred_element_type=jnp.float32) + bp_ref[...]


def kernel(x, eigvecs, eigvals, eig_mask, W1, b1, W2, b2, Wp, bp):
    ntiles = N // TN
    evals_col = eigvals.reshape(K, 1)
    mask_col = eig_mask.astype(jnp.float32).reshape(K, 1)
    w1t = W1.T
    b1_col = b1.reshape(K // 2, 1)
    w2t = W2.T
    b2_col = b2.reshape(K, 1)
    bp_row = bp.reshape(1, OUT)

    parts = pl.pallas_call(
        _pass1,
        grid=(CORES, ntiles // CORES),
        in_specs=[
            pl.BlockSpec((TN, D), lambda c, j: (c * (N // TN // CORES) + j, 0)),
            pl.BlockSpec((TN, K), lambda c, j: (c * (N // TN // CORES) + j, 0)),
        ],
        out_specs=pl.BlockSpec((1, K, D), lambda c, j: (c, 0, 0)),
        out_shape=jax.ShapeDtypeStruct((CORES, K, D), jnp.float32),
        compiler_params=pltpu.CompilerParams(
            dimension_semantics=("parallel", "arbitrary")),
    )(x, eigvecs)

    out = pl.pallas_call(
        _pass2,
        grid=(ntiles,),
        in_specs=[
            pl.BlockSpec((CORES, K, D), lambda i: (0, 0, 0)),
            pl.BlockSpec((K, 1), lambda i: (0, 0)),
            pl.BlockSpec((K, 1), lambda i: (0, 0)),
            pl.BlockSpec((K // 2, K), lambda i: (0, 0)),
            pl.BlockSpec((K // 2, 1), lambda i: (0, 0)),
            pl.BlockSpec((K, K // 2), lambda i: (0, 0)),
            pl.BlockSpec((K, 1), lambda i: (0, 0)),
            pl.BlockSpec((D, OUT), lambda i: (0, 0)),
            pl.BlockSpec((TN, K), lambda i: (i, 0)),
            pl.BlockSpec((1, OUT), lambda i: (0, 0)),
        ],
        out_specs=pl.BlockSpec((TN, OUT), lambda i: (i, 0)),
        out_shape=jax.ShapeDtypeStruct((N, OUT), jnp.float32),
        compiler_params=pltpu.CompilerParams(
            dimension_semantics=("parallel",)),
    )(parts, evals_col, mask_col, w1t, b1_col, w2t, b2_col, Wp, eigvecs,
      bp_row)
    return out
